# Initial kernel scaffold; baseline (speedup 1.0000x reference)
import jax
import jax.numpy as jnp
from jax.experimental import pallas as pl


def kernel(positions, cos_cached, sin_cached):
    raise NotImplementedError("write your pallas kernel here")


# speedup vs baseline: 4.8593x; 4.8593x over previous
import functools

import jax
import jax.numpy as jnp
from jax import lax
from jax.experimental import pallas as pl
from jax.experimental.pallas import tpu as pltpu
from jax.experimental.pallas import tpu_sc as plsc

_NC = 2
_NS = 16
_NW = _NC * _NS
_D = 128
_B = 4 * 8192
_BP = _B // _NW
_C = 128
_NCH = _BP // _C


@functools.partial(
    pl.kernel,
    mesh=plsc.VectorSubcoreMesh(core_axis_name="c", subcore_axis_name="s"),
    out_type=[
        jax.ShapeDtypeStruct((_B, _D), jnp.float32),
        jax.ShapeDtypeStruct((_B, _D), jnp.float32),
    ],
    scratch_types=[
        pltpu.VMEM((_NCH, _C), jnp.int32),
        pltpu.VMEM((_C, _D), jnp.float32),
        pltpu.VMEM((_C, _D), jnp.float32),
        pltpu.SemaphoreType.DMA,
        pltpu.SemaphoreType.DMA,
    ],
)
def _rope_gather(pos_hbm, cos_hbm, sin_hbm, cos_out, sin_out,
                 idx_v, cos_v, sin_v, sem_c, sem_s):
    wid = lax.axis_index("s") * _NC + lax.axis_index("c")
    base = wid * _BP
    pltpu.sync_copy(pos_hbm.at[wid], idx_v)
    for c in range(_NCH):
        gc = pltpu.async_copy(cos_hbm.at[idx_v.at[c]], cos_v, sem_c)
        gs = pltpu.async_copy(sin_hbm.at[idx_v.at[c]], sin_v, sem_s)
        gc.wait()
        gs.wait()
        pltpu.sync_copy(cos_v, cos_out.at[pl.ds(base + c * _C, _C)])
        pltpu.sync_copy(sin_v, sin_out.at[pl.ds(base + c * _C, _C)])


def kernel(positions, cos_cached, sin_cached):
    shape = positions.shape
    pos = positions.reshape(_NW, _NCH, _C)
    cos, sin = _rope_gather(pos, cos_cached, sin_cached)
    return (cos.reshape(*shape, _D), sin.reshape(*shape, _D))


# speedup vs baseline: 5.1359x; 1.0569x over previous
import functools

import jax
import jax.numpy as jnp
from jax import lax
from jax.experimental import pallas as pl
from jax.experimental.pallas import tpu as pltpu
from jax.experimental.pallas import tpu_sc as plsc

_NC = 2
_NS = 16
_NW = _NC * _NS
_D = 128
_B = 4 * 8192
_BP = _B // _NW
_C = 128
_NCH = _BP // _C


@functools.partial(
    pl.kernel,
    mesh=plsc.VectorSubcoreMesh(core_axis_name="c", subcore_axis_name="s"),
    out_type=[
        jax.ShapeDtypeStruct((_B, _D), jnp.float32),
        jax.ShapeDtypeStruct((_B, _D), jnp.float32),
    ],
    scratch_types=[
        pltpu.VMEM((_NCH, _C), jnp.int32),
        pltpu.VMEM((_C, _D), jnp.float32),
        pltpu.VMEM((_C, _D), jnp.float32),
        pltpu.VMEM((_C, _D), jnp.float32),
        pltpu.VMEM((_C, _D), jnp.float32),
        pltpu.SemaphoreType.DMA,
        pltpu.SemaphoreType.DMA,
        pltpu.SemaphoreType.DMA,
        pltpu.SemaphoreType.DMA,
    ],
)
def _rope_gather(pos_hbm, cos_hbm, sin_hbm, cos_out, sin_out,
                 idx_v, cos0, cos1, sin0, sin1, sg0, sg1, sw0, sw1):
    wid = lax.axis_index("s") * _NC + lax.axis_index("c")
    base = wid * _BP
    pltpu.sync_copy(pos_hbm.at[wid], idx_v)
    cosb, sinb = (cos0, cos1), (sin0, sin1)
    sg, sw = (sg0, sg1), (sw0, sw1)
    gh, wh = {}, {}
    for c in range(_NCH):
        b = c % 2
        if c >= 2:
            for h in wh[c - 2]:
                h.wait()
        gh[c] = (pltpu.async_copy(cos_hbm.at[idx_v.at[c]], cosb[b], sg[b]),
                 pltpu.async_copy(sin_hbm.at[idx_v.at[c]], sinb[b], sg[b]))
        if c >= 1:
            p, pb = c - 1, (c - 1) % 2
            for h in gh[p]:
                h.wait()
            off = base + p * _C
            wh[p] = (pltpu.async_copy(cosb[pb], cos_out.at[pl.ds(off, _C)], sw[pb]),
                     pltpu.async_copy(sinb[pb], sin_out.at[pl.ds(off, _C)], sw[pb]))
    last, lb = _NCH - 1, (_NCH - 1) % 2
    for h in gh[last]:
        h.wait()
    off = base + last * _C
    wh[last] = (pltpu.async_copy(cosb[lb], cos_out.at[pl.ds(off, _C)], sw[lb]),
                pltpu.async_copy(sinb[lb], sin_out.at[pl.ds(off, _C)], sw[lb]))
    for c in (last - 1, last):
        for h in wh[c]:
            h.wait()


def kernel(positions, cos_cached, sin_cached):
    shape = positions.shape
    pos = positions.reshape(_NW, _NCH, _C)
    cos, sin = _rope_gather(pos, cos_cached, sin_cached)
    return (cos.reshape(*shape, _D), sin.reshape(*shape, _D))
